# baseline (speedup 1.0000x reference)
import functools

import jax
import jax.numpy as jnp
from jax import lax
from jax.experimental import pallas as pl
from jax.experimental.pallas import tpu as pltpu
from jax.experimental.pallas import tpu_sc as plsc

N = 100000
D_ATOM = 32
D_RES = 16
D_NUM = 16
D_OUT = D_ATOM + D_RES + D_NUM

NC, NS = 2, 16
NW = NC * NS
SUB = 80
NSUB = 5
C = SUB * NSUB
NG = C // 16
NCHUNKS = N // C
BIG = NCHUNKS - NW * (NCHUNKS // NW)


def _body(atom_idx_hbm, res_idx_hbm, featsT_hbm, wa_hbm, wr_hbm, outT_hbm,
          idx_a, idx_r, rows_a, rows_r, featsT_v, outT_v, sem):
    wid = lax.axis_index("s") * NC + lax.axis_index("c")
    base_chunks = NCHUNKS // NW
    start = wid * base_chunks + jnp.minimum(wid, BIG)
    count = jnp.where(wid < BIG, base_chunks + 1, base_chunks)
    lanes = lax.iota(jnp.int32, 16)

    def chunk_body(chunk, carry):
        base = chunk * C
        pltpu.sync_copy(atom_idx_hbm.at[pl.ds(base, C)], idx_a)
        pltpu.sync_copy(res_idx_hbm.at[pl.ds(base, C)], idx_r)
        pltpu.sync_copy(featsT_hbm.at[:, pl.ds(base, C)], featsT_v)
        copies = []
        for j in range(NSUB):
            copies.append(
                pltpu.async_copy(wa_hbm.at[idx_a.at[pl.ds(SUB * j, SUB)]],
                                 rows_a.at[pl.ds(SUB * j, SUB)], sem))
            copies.append(
                pltpu.async_copy(wr_hbm.at[idx_r.at[pl.ds(SUB * j, SUB)]],
                                 rows_r.at[pl.ds(SUB * j, SUB)], sem))
        for cp in copies:
            cp.wait()

        def col_group(g, carry2):
            rows16 = g * 16 + lanes
            for c in range(D_ATOM):
                cols16 = jnp.full((16,), c, jnp.int32)
                outT_v[c, pl.ds(g * 16, 16)] = plsc.load_gather(
                    rows_a, [rows16, cols16])
            for c in range(D_RES):
                cols16 = jnp.full((16,), c, jnp.int32)
                outT_v[D_ATOM + c, pl.ds(g * 16, 16)] = plsc.load_gather(
                    rows_r, [rows16, cols16])
            for c in range(D_NUM):
                outT_v[D_ATOM + D_RES + c, pl.ds(g * 16, 16)] = \
                    featsT_v[c, pl.ds(g * 16, 16)]
            return carry2

        lax.fori_loop(0, NG, col_group, 0)
        pltpu.sync_copy(outT_v, outT_hbm.at[:, pl.ds(base, C)])
        return carry

    lax.fori_loop(start, start + count, chunk_body, 0)


@jax.jit
def _sc_embed(atom_types, residue_types, featsT, W_atom, W_res):
    mesh = plsc.VectorSubcoreMesh(core_axis_name="c", subcore_axis_name="s",
                                  num_cores=NC, num_subcores=NS)
    f = functools.partial(
        pl.kernel,
        out_type=jax.ShapeDtypeStruct((D_OUT, N), jnp.float32),
        mesh=mesh,
        scratch_types=[
            pltpu.VMEM((C,), jnp.int32),
            pltpu.VMEM((C,), jnp.int32),
            pltpu.VMEM((C, D_ATOM), jnp.float32),
            pltpu.VMEM((C, D_RES), jnp.float32),
            pltpu.VMEM((D_NUM, C), jnp.float32),
            pltpu.VMEM((D_OUT, C), jnp.float32),
            pltpu.SemaphoreType.DMA,
        ],
        compiler_params=pltpu.CompilerParams(use_tc_tiling_on_sc=False,
                                             needs_layout_passes=False),
    )(_body)
    return f(atom_types, residue_types, featsT, W_atom, W_res)


def kernel(atom_types, residue_types, extra_feats, W_atom, W_res):
    outT = _sc_embed(atom_types, residue_types, extra_feats.T, W_atom, W_res)
    return outT.T


# speedup vs baseline: 1.1673x; 1.1673x over previous
import functools

import jax
import jax.numpy as jnp
from jax import lax
from jax.experimental import pallas as pl
from jax.experimental.pallas import tpu as pltpu
from jax.experimental.pallas import tpu_sc as plsc

N = 100000
D_ATOM = 32
D_RES = 16
D_NUM = 16
D_OUT = D_ATOM + D_RES + D_NUM

NC, NS = 2, 16
NW = NC * NS
SUB = 80
NSUB = 5
C = SUB * NSUB
NG = C // 16
NCHUNKS = N // C
BIG = NCHUNKS - NW * (NCHUNKS // NW)


def _body(atom_idx_hbm, res_idx_hbm, featsT_hbm, wa_hbm, wr_hbm, outT_hbm,
          idx_a, idx_r, rows_a, rows_r, featsT_v, outT_v, sem):
    wid = lax.axis_index("s") * NC + lax.axis_index("c")
    base_chunks = NCHUNKS // NW
    start = wid * base_chunks + jnp.minimum(wid, BIG)
    count = jnp.where(wid < BIG, base_chunks + 1, base_chunks)
    lanes = lax.iota(jnp.int32, 16)

    def chunk_body(chunk, carry):
        base = chunk * C
        pltpu.sync_copy(atom_idx_hbm.at[pl.ds(base, C)], idx_a)
        pltpu.sync_copy(res_idx_hbm.at[pl.ds(base, C)], idx_r)
        pltpu.sync_copy(featsT_hbm.at[:, pl.ds(base, C)], featsT_v)
        copies = []
        for j in range(NSUB):
            copies.append(
                pltpu.async_copy(wa_hbm.at[idx_a.at[pl.ds(SUB * j, SUB)]],
                                 rows_a.at[pl.ds(SUB * j, SUB)], sem))
            copies.append(
                pltpu.async_copy(wr_hbm.at[idx_r.at[pl.ds(SUB * j, SUB)]],
                                 rows_r.at[pl.ds(SUB * j, SUB)], sem))
        for cp in copies:
            cp.wait()

        @plsc.parallel_loop(0, NG)
        def col_group(g):
            rows16 = g * 16 + lanes
            for c in range(D_ATOM):
                cols16 = jnp.full((16,), c, jnp.int32)
                outT_v[c, pl.ds(g * 16, 16)] = plsc.load_gather(
                    rows_a, [rows16, cols16])
            for c in range(D_RES):
                cols16 = jnp.full((16,), c, jnp.int32)
                outT_v[D_ATOM + c, pl.ds(g * 16, 16)] = plsc.load_gather(
                    rows_r, [rows16, cols16])
            for c in range(D_NUM):
                outT_v[D_ATOM + D_RES + c, pl.ds(g * 16, 16)] = \
                    featsT_v[c, pl.ds(g * 16, 16)]
        pltpu.sync_copy(outT_v, outT_hbm.at[:, pl.ds(base, C)])
        return carry

    lax.fori_loop(start, start + count, chunk_body, 0)


@jax.jit
def _sc_embed(atom_types, residue_types, featsT, W_atom, W_res):
    mesh = plsc.VectorSubcoreMesh(core_axis_name="c", subcore_axis_name="s",
                                  num_cores=NC, num_subcores=NS)
    f = functools.partial(
        pl.kernel,
        out_type=jax.ShapeDtypeStruct((D_OUT, N), jnp.float32),
        mesh=mesh,
        scratch_types=[
            pltpu.VMEM((C,), jnp.int32),
            pltpu.VMEM((C,), jnp.int32),
            pltpu.VMEM((C, D_ATOM), jnp.float32),
            pltpu.VMEM((C, D_RES), jnp.float32),
            pltpu.VMEM((D_NUM, C), jnp.float32),
            pltpu.VMEM((D_OUT, C), jnp.float32),
            pltpu.SemaphoreType.DMA,
        ],
        compiler_params=pltpu.CompilerParams(use_tc_tiling_on_sc=False,
                                             needs_layout_passes=False),
    )(_body)
    return f(atom_types, residue_types, featsT, W_atom, W_res)


def kernel(atom_types, residue_types, extra_feats, W_atom, W_res):
    outT = _sc_embed(atom_types, residue_types, extra_feats.T, W_atom, W_res)
    return outT.T


# speedup vs baseline: 1.2603x; 1.0797x over previous
import functools

import jax
import jax.numpy as jnp
from jax import lax
from jax.experimental import pallas as pl
from jax.experimental.pallas import tpu as pltpu
from jax.experimental.pallas import tpu_sc as plsc

N = 100000
D_ATOM = 32
D_RES = 16
D_NUM = 16
D_OUT = D_ATOM + D_RES + D_NUM

NC, NS = 2, 16
NW = NC * NS
SUB = 80
NSUB = 5
C = SUB * NSUB
NG = C // 16
NCHUNKS = N // C
BIG = NCHUNKS - NW * (NCHUNKS // NW)


def _body(atom_idx_hbm, res_idx_hbm, featsT_hbm, wa_hbm, wr_hbm, outT_hbm,
          bufs0, bufs1, sem0, sem1):
    wid = lax.axis_index("s") * NC + lax.axis_index("c")
    base_chunks = NCHUNKS // NW
    start = wid * base_chunks + jnp.minimum(wid, BIG)
    count = jnp.where(wid < BIG, base_chunks + 1, base_chunks)
    end = start + count
    lanes = lax.iota(jnp.int32, 16)

    def fire(k, bufs, sem):
        idx_a, idx_r, rows_a, rows_r, featsT_v, _ = bufs
        base = k * C
        pltpu.sync_copy(atom_idx_hbm.at[pl.ds(base, C)], idx_a)
        pltpu.sync_copy(res_idx_hbm.at[pl.ds(base, C)], idx_r)
        pltpu.async_copy(featsT_hbm.at[:, pl.ds(base, C)], featsT_v, sem)
        for j in range(NSUB):
            pltpu.async_copy(wa_hbm.at[idx_a.at[pl.ds(SUB * j, SUB)]],
                             rows_a.at[pl.ds(SUB * j, SUB)], sem)
            pltpu.async_copy(wr_hbm.at[idx_r.at[pl.ds(SUB * j, SUB)]],
                             rows_r.at[pl.ds(SUB * j, SUB)], sem)

    def drain(bufs, sem):
        _, _, rows_a, rows_r, featsT_v, _ = bufs
        pltpu.make_async_copy(featsT_hbm.at[:, pl.ds(0, C)], featsT_v,
                              sem).wait()
        for j in range(NSUB):
            pltpu.make_async_copy(wa_hbm.at[pl.ds(0, SUB)],
                                  rows_a.at[pl.ds(SUB * j, SUB)], sem).wait()
            pltpu.make_async_copy(wr_hbm.at[pl.ds(0, SUB)],
                                  rows_r.at[pl.ds(SUB * j, SUB)], sem).wait()

    def proc(k, bufs):
        _, _, rows_a, rows_r, featsT_v, outT_v = bufs
        base = k * C

        @plsc.parallel_loop(0, NG)
        def col_group(g):
            rows16 = g * 16 + lanes
            for c in range(D_ATOM):
                cols16 = jnp.full((16,), c, jnp.int32)
                outT_v[c, pl.ds(g * 16, 16)] = plsc.load_gather(
                    rows_a, [rows16, cols16])
            for c in range(D_RES):
                cols16 = jnp.full((16,), c, jnp.int32)
                outT_v[D_ATOM + c, pl.ds(g * 16, 16)] = plsc.load_gather(
                    rows_r, [rows16, cols16])
            for c in range(D_NUM):
                outT_v[D_ATOM + D_RES + c, pl.ds(g * 16, 16)] = \
                    featsT_v[c, pl.ds(g * 16, 16)]

        pltpu.sync_copy(outT_v, outT_hbm.at[:, pl.ds(base, C)])

    fire(start, bufs0, sem0)

    def pair_body(p, carry):
        k0 = start + 2 * p
        k1 = k0 + 1

        @pl.when(k1 < end)
        def _():
            fire(k1, bufs1, sem1)

        drain(bufs0, sem0)
        proc(k0, bufs0)

        @pl.when(k1 < end)
        def _():
            @pl.when(k1 + 1 < end)
            def _():
                fire(k1 + 1, bufs0, sem0)

            drain(bufs1, sem1)
            proc(k1, bufs1)

        return carry

    lax.fori_loop(0, (base_chunks + 2) // 2, pair_body, 0)


def _buf_types():
    return (
        pltpu.VMEM((C,), jnp.int32),
        pltpu.VMEM((C,), jnp.int32),
        pltpu.VMEM((C, D_ATOM), jnp.float32),
        pltpu.VMEM((C, D_RES), jnp.float32),
        pltpu.VMEM((D_NUM, C), jnp.float32),
        pltpu.VMEM((D_OUT, C), jnp.float32),
    )


@jax.jit
def _sc_embed(atom_types, residue_types, featsT, W_atom, W_res):
    mesh = plsc.VectorSubcoreMesh(core_axis_name="c", subcore_axis_name="s",
                                  num_cores=NC, num_subcores=NS)
    f = functools.partial(
        pl.kernel,
        out_type=jax.ShapeDtypeStruct((D_OUT, N), jnp.float32),
        mesh=mesh,
        scratch_types=[
            _buf_types(),
            _buf_types(),
            pltpu.SemaphoreType.DMA,
            pltpu.SemaphoreType.DMA,
        ],
        compiler_params=pltpu.CompilerParams(use_tc_tiling_on_sc=False,
                                             needs_layout_passes=False),
    )(_body)
    return f(atom_types, residue_types, featsT, W_atom, W_res)


def kernel(atom_types, residue_types, extra_feats, W_atom, W_res):
    outT = _sc_embed(atom_types, residue_types, extra_feats.T, W_atom, W_res)
    return outT.T


# speedup vs baseline: 1.3006x; 1.0320x over previous
import functools

import jax
import jax.numpy as jnp
from jax import lax
from jax.experimental import pallas as pl
from jax.experimental.pallas import tpu as pltpu
from jax.experimental.pallas import tpu_sc as plsc

N = 100000
D_ATOM = 32
D_RES = 16
D_AR = D_ATOM + D_RES

NC, NS = 2, 16
NW = NC * NS
SUB = 80
NSUB = 5
C = SUB * NSUB
NG = C // 16
NCHUNKS = N // C
BIG = NCHUNKS - NW * (NCHUNKS // NW)


def _body(atom_idx_hbm, res_idx_hbm, wa_hbm, wr_hbm, outT_hbm,
          bufs0, bufs1, sem0, sem1):
    wid = lax.axis_index("s") * NC + lax.axis_index("c")
    base_chunks = NCHUNKS // NW
    start = wid * base_chunks + jnp.minimum(wid, BIG)
    count = jnp.where(wid < BIG, base_chunks + 1, base_chunks)
    end = start + count
    lanes = lax.iota(jnp.int32, 16)

    def fire(k, bufs, sem):
        idx_a, idx_r, rows_a, rows_r, _ = bufs
        base = k * C
        pltpu.sync_copy(atom_idx_hbm.at[pl.ds(base, C)], idx_a)
        pltpu.sync_copy(res_idx_hbm.at[pl.ds(base, C)], idx_r)
        for j in range(NSUB):
            pltpu.async_copy(wa_hbm.at[idx_a.at[pl.ds(SUB * j, SUB)]],
                             rows_a.at[pl.ds(SUB * j, SUB)], sem)
            pltpu.async_copy(wr_hbm.at[idx_r.at[pl.ds(SUB * j, SUB)]],
                             rows_r.at[pl.ds(SUB * j, SUB)], sem)

    def drain(bufs, sem):
        _, _, rows_a, rows_r, _ = bufs
        for j in range(NSUB):
            pltpu.make_async_copy(wa_hbm.at[pl.ds(0, SUB)],
                                  rows_a.at[pl.ds(SUB * j, SUB)], sem).wait()
            pltpu.make_async_copy(wr_hbm.at[pl.ds(0, SUB)],
                                  rows_r.at[pl.ds(SUB * j, SUB)], sem).wait()

    def proc(k, bufs):
        _, _, rows_a, rows_r, outT_v = bufs
        base = k * C

        @plsc.parallel_loop(0, NG)
        def col_group(g):
            rows16 = g * 16 + lanes
            for c in range(D_ATOM):
                cols16 = jnp.full((16,), c, jnp.int32)
                outT_v[c, pl.ds(g * 16, 16)] = plsc.load_gather(
                    rows_a, [rows16, cols16])
            for c in range(D_RES):
                cols16 = jnp.full((16,), c, jnp.int32)
                outT_v[D_ATOM + c, pl.ds(g * 16, 16)] = plsc.load_gather(
                    rows_r, [rows16, cols16])

        pltpu.sync_copy(outT_v, outT_hbm.at[:, pl.ds(base, C)])

    fire(start, bufs0, sem0)

    def pair_body(p, carry):
        k0 = start + 2 * p
        k1 = k0 + 1

        @pl.when(k1 < end)
        def _():
            fire(k1, bufs1, sem1)

        drain(bufs0, sem0)
        proc(k0, bufs0)

        @pl.when(k1 < end)
        def _():
            @pl.when(k1 + 1 < end)
            def _():
                fire(k1 + 1, bufs0, sem0)

            drain(bufs1, sem1)
            proc(k1, bufs1)

        return carry

    lax.fori_loop(0, (base_chunks + 2) // 2, pair_body, 0)


def _buf_types():
    return (
        pltpu.VMEM((C,), jnp.int32),
        pltpu.VMEM((C,), jnp.int32),
        pltpu.VMEM((C, D_ATOM), jnp.float32),
        pltpu.VMEM((C, D_RES), jnp.float32),
        pltpu.VMEM((D_AR, C), jnp.float32),
    )


@jax.jit
def _sc_embed(atom_types, residue_types, W_atom, W_res):
    mesh = plsc.VectorSubcoreMesh(core_axis_name="c", subcore_axis_name="s",
                                  num_cores=NC, num_subcores=NS)
    f = functools.partial(
        pl.kernel,
        out_type=jax.ShapeDtypeStruct((D_AR, N), jnp.float32),
        mesh=mesh,
        scratch_types=[
            _buf_types(),
            _buf_types(),
            pltpu.SemaphoreType.DMA,
            pltpu.SemaphoreType.DMA,
        ],
        compiler_params=pltpu.CompilerParams(use_tc_tiling_on_sc=False,
                                             needs_layout_passes=False),
    )(_body)
    return f(atom_types, residue_types, W_atom, W_res)


def kernel(atom_types, residue_types, extra_feats, W_atom, W_res):
    outT_ar = _sc_embed(atom_types, residue_types, W_atom, W_res)
    return jnp.concatenate([outT_ar, extra_feats.T], axis=0).T


# speedup vs baseline: 1.3627x; 1.0478x over previous
import functools

import jax
import jax.numpy as jnp
from jax import lax
from jax.experimental import pallas as pl
from jax.experimental.pallas import tpu as pltpu
from jax.experimental.pallas import tpu_sc as plsc

N = 100000
D_ATOM = 32
D_RES = 16
D_AR = D_ATOM + D_RES

NC, NS = 2, 16
NW = NC * NS
SUB = 80
NSUB = 5
C = SUB * NSUB
NG = C // 16
NCHUNKS = N // C
BASE_CHUNKS = NCHUNKS // NW
BIG = NCHUNKS - NW * BASE_CHUNKS
MAXC = BASE_CHUNKS + 1


def _body(atom_idx_hbm, res_idx_hbm, wa_hbm, wr_hbm, outT_hbm,
          idx_a, idx_r, bufs0, bufs1, sem0, sem1, semw0, semw1):
    wid = lax.axis_index("s") * NC + lax.axis_index("c")
    start = wid * BASE_CHUNKS + jnp.minimum(wid, BIG)
    count = jnp.where(wid < BIG, BASE_CHUNKS + 1, BASE_CHUNKS)
    end = start + count
    base0 = start * C
    lanes = lax.iota(jnp.int32, 16)

    pltpu.sync_copy(atom_idx_hbm.at[pl.ds(base0, BASE_CHUNKS * C)],
                    idx_a.at[pl.ds(0, BASE_CHUNKS * C)])
    pltpu.sync_copy(res_idx_hbm.at[pl.ds(base0, BASE_CHUNKS * C)],
                    idx_r.at[pl.ds(0, BASE_CHUNKS * C)])

    @pl.when(count == MAXC)
    def _():
        pltpu.sync_copy(
            atom_idx_hbm.at[pl.ds(base0 + BASE_CHUNKS * C, C)],
            idx_a.at[pl.ds(BASE_CHUNKS * C, C)])
        pltpu.sync_copy(
            res_idx_hbm.at[pl.ds(base0 + BASE_CHUNKS * C, C)],
            idx_r.at[pl.ds(BASE_CHUNKS * C, C)])

    def fire(k, bufs, sem):
        rows_a, rows_r, _ = bufs
        off = (k - start) * C
        for j in range(NSUB):
            pltpu.async_copy(wa_hbm.at[idx_a.at[pl.ds(off + SUB * j, SUB)]],
                             rows_a.at[pl.ds(SUB * j, SUB)], sem)
            pltpu.async_copy(wr_hbm.at[idx_r.at[pl.ds(off + SUB * j, SUB)]],
                             rows_r.at[pl.ds(SUB * j, SUB)], sem)

    def drain(bufs, sem):
        rows_a, rows_r, _ = bufs
        for j in range(NSUB):
            pltpu.make_async_copy(wa_hbm.at[pl.ds(0, SUB)],
                                  rows_a.at[pl.ds(SUB * j, SUB)], sem).wait()
            pltpu.make_async_copy(wr_hbm.at[pl.ds(0, SUB)],
                                  rows_r.at[pl.ds(SUB * j, SUB)], sem).wait()

    def proc(k, bufs, semw):
        rows_a, rows_r, outT_v = bufs
        base = k * C

        @pl.when(k >= start + 2)
        def _():
            pltpu.make_async_copy(outT_v, outT_hbm.at[:, pl.ds(0, C)],
                                  semw).wait()

        @plsc.parallel_loop(0, NG)
        def col_group(g):
            rows16 = g * 16 + lanes
            for c in range(D_ATOM):
                cols16 = jnp.full((16,), c, jnp.int32)
                outT_v[c, pl.ds(g * 16, 16)] = plsc.load_gather(
                    rows_a, [rows16, cols16])
            for c in range(D_RES):
                cols16 = jnp.full((16,), c, jnp.int32)
                outT_v[D_ATOM + c, pl.ds(g * 16, 16)] = plsc.load_gather(
                    rows_r, [rows16, cols16])

        pltpu.async_copy(outT_v, outT_hbm.at[:, pl.ds(base, C)], semw)

    fire(start, bufs0, sem0)

    def pair_body(p, carry):
        k0 = start + 2 * p
        k1 = k0 + 1

        @pl.when(k1 < end)
        def _():
            fire(k1, bufs1, sem1)

        drain(bufs0, sem0)
        proc(k0, bufs0, semw0)

        @pl.when(k1 < end)
        def _():
            @pl.when(k1 + 1 < end)
            def _():
                fire(k1 + 1, bufs0, sem0)

            drain(bufs1, sem1)
            proc(k1, bufs1, semw1)

        return carry

    lax.fori_loop(0, (MAXC + 1) // 2, pair_body, 0)

    pltpu.make_async_copy(bufs0[2], outT_hbm.at[:, pl.ds(0, C)],
                          semw0).wait()
    pltpu.make_async_copy(bufs1[2], outT_hbm.at[:, pl.ds(0, C)],
                          semw1).wait()


def _buf_types():
    return (
        pltpu.VMEM((C, D_ATOM), jnp.float32),
        pltpu.VMEM((C, D_RES), jnp.float32),
        pltpu.VMEM((D_AR, C), jnp.float32),
    )


@jax.jit
def _sc_embed(atom_types, residue_types, W_atom, W_res):
    mesh = plsc.VectorSubcoreMesh(core_axis_name="c", subcore_axis_name="s",
                                  num_cores=NC, num_subcores=NS)
    f = functools.partial(
        pl.kernel,
        out_type=jax.ShapeDtypeStruct((D_AR, N), jnp.float32),
        mesh=mesh,
        scratch_types=[
            pltpu.VMEM((MAXC * C,), jnp.int32),
            pltpu.VMEM((MAXC * C,), jnp.int32),
            _buf_types(),
            _buf_types(),
            pltpu.SemaphoreType.DMA,
            pltpu.SemaphoreType.DMA,
            pltpu.SemaphoreType.DMA,
            pltpu.SemaphoreType.DMA,
        ],
        compiler_params=pltpu.CompilerParams(use_tc_tiling_on_sc=False,
                                             needs_layout_passes=False),
    )(_body)
    return f(atom_types, residue_types, W_atom, W_res)


def kernel(atom_types, residue_types, extra_feats, W_atom, W_res):
    outT_ar = _sc_embed(atom_types, residue_types, W_atom, W_res)
    return jnp.concatenate([outT_ar, extra_feats.T], axis=0).T


# speedup vs baseline: 1.3667x; 1.0029x over previous
import functools

import jax
import jax.numpy as jnp
from jax import lax
from jax.experimental import pallas as pl
from jax.experimental.pallas import tpu as pltpu
from jax.experimental.pallas import tpu_sc as plsc

N = 100000
D_ATOM = 32
D_RES = 16
D_AR = D_ATOM + D_RES

NC, NS = 2, 16
NW = NC * NS
SUB = 80
NSUB = 5
C = SUB * NSUB
NG = C // 16
NCHUNKS = N // C
BASE_CHUNKS = NCHUNKS // NW
BIG = NCHUNKS - NW * BASE_CHUNKS
MAXC = BASE_CHUNKS + 1


def _body(atom_idx_hbm, res_idx_hbm, wa_hbm, wr_hbm, outT_hbm,
          idx_a, idx_r, bufs0, bufs1, sem0, sem1, semw0, semw1):
    wid = lax.axis_index("s") * NC + lax.axis_index("c")
    start = wid * BASE_CHUNKS + jnp.minimum(wid, BIG)
    count = jnp.where(wid < BIG, BASE_CHUNKS + 1, BASE_CHUNKS)
    end = start + count
    base0 = start * C
    lanes = lax.iota(jnp.int32, 16)

    pltpu.sync_copy(atom_idx_hbm.at[pl.ds(base0, BASE_CHUNKS * C)],
                    idx_a.at[pl.ds(0, BASE_CHUNKS * C)])
    pltpu.sync_copy(res_idx_hbm.at[pl.ds(base0, BASE_CHUNKS * C)],
                    idx_r.at[pl.ds(0, BASE_CHUNKS * C)])

    @pl.when(count == MAXC)
    def _():
        pltpu.sync_copy(
            atom_idx_hbm.at[pl.ds(base0 + BASE_CHUNKS * C, C)],
            idx_a.at[pl.ds(BASE_CHUNKS * C, C)])
        pltpu.sync_copy(
            res_idx_hbm.at[pl.ds(base0 + BASE_CHUNKS * C, C)],
            idx_r.at[pl.ds(BASE_CHUNKS * C, C)])

    def fire(k, bufs, sem):
        rows_a, rows_r, _ = bufs
        off = (k - start) * C
        for j in range(NSUB):
            pltpu.async_copy(wa_hbm.at[idx_a.at[pl.ds(off + SUB * j, SUB)]],
                             rows_a.at[pl.ds(SUB * j, SUB)], sem)
            pltpu.async_copy(wr_hbm.at[idx_r.at[pl.ds(off + SUB * j, SUB)]],
                             rows_r.at[pl.ds(SUB * j, SUB)], sem)

    def drain(bufs, sem):
        rows_a, rows_r, _ = bufs
        for j in range(NSUB):
            pltpu.make_async_copy(wa_hbm.at[pl.ds(0, SUB)],
                                  rows_a.at[pl.ds(SUB * j, SUB)], sem).wait()
            pltpu.make_async_copy(wr_hbm.at[pl.ds(0, SUB)],
                                  rows_r.at[pl.ds(SUB * j, SUB)], sem).wait()

    def proc(k, bufs, semw):
        rows_a, rows_r, outT_v = bufs
        base = k * C

        @pl.when(k >= start + 2)
        def _():
            pltpu.make_async_copy(outT_v, outT_hbm.at[:, pl.ds(0, C)],
                                  semw).wait()

        @plsc.parallel_loop(0, NG)
        def col_group(g):
            rows16 = g * 16 + lanes
            for c in range(D_ATOM):
                cols16 = jnp.full((16,), c, jnp.int32)
                outT_v[c, pl.ds(g * 16, 16)] = plsc.load_gather(
                    rows_a, [rows16, cols16])
            for c in range(D_RES):
                cols16 = jnp.full((16,), c, jnp.int32)
                outT_v[D_ATOM + c, pl.ds(g * 16, 16)] = plsc.load_gather(
                    rows_r, [rows16, cols16])

        pltpu.async_copy(outT_v, outT_hbm.at[:, pl.ds(base, C)], semw)

    fire(start, bufs0, sem0)

    def pair_body(p, carry):
        k0 = start + 2 * p
        k1 = k0 + 1

        @pl.when(k1 < end)
        def _():
            fire(k1, bufs1, sem1)

        drain(bufs0, sem0)
        proc(k0, bufs0, semw0)

        @pl.when(k1 < end)
        def _():
            @pl.when(k1 + 1 < end)
            def _():
                fire(k1 + 1, bufs0, sem0)

            drain(bufs1, sem1)
            proc(k1, bufs1, semw1)

        return carry

    lax.fori_loop(0, (MAXC + 1) // 2, pair_body, 0)

    pltpu.make_async_copy(bufs0[2], outT_hbm.at[:, pl.ds(0, C)],
                          semw0).wait()
    pltpu.make_async_copy(bufs1[2], outT_hbm.at[:, pl.ds(0, C)],
                          semw1).wait()


def _buf_types():
    return (
        pltpu.VMEM((C, D_ATOM), jnp.float32),
        pltpu.VMEM((C, D_RES), jnp.float32),
        pltpu.VMEM((D_AR, C), jnp.float32),
    )


@jax.jit
def _sc_embed(atom_types, residue_types, W_atom, W_res):
    mesh = plsc.VectorSubcoreMesh(core_axis_name="c", subcore_axis_name="s",
                                  num_cores=NC, num_subcores=NS)
    f = functools.partial(
        pl.kernel,
        out_type=jax.ShapeDtypeStruct((D_AR, N), jnp.float32),
        mesh=mesh,
        scratch_types=[
            pltpu.VMEM((MAXC * C,), jnp.int32),
            pltpu.VMEM((MAXC * C,), jnp.int32),
            _buf_types(),
            _buf_types(),
            pltpu.SemaphoreType.DMA,
            pltpu.SemaphoreType.DMA,
            pltpu.SemaphoreType.DMA,
            pltpu.SemaphoreType.DMA,
        ],
        compiler_params=pltpu.CompilerParams(use_tc_tiling_on_sc=False,
                                             needs_layout_passes=False),
    )(_body)
    return f(atom_types, residue_types, W_atom, W_res)


def kernel(atom_types, residue_types, extra_feats, W_atom, W_res):
    outT_ar = _sc_embed(atom_types, residue_types, W_atom, W_res)
    return jnp.concatenate([outT_ar.T, extra_feats], axis=1)


# speedup vs baseline: 1.4808x; 1.0835x over previous
import functools

import jax
import jax.numpy as jnp
from jax import lax
from jax.experimental import pallas as pl
from jax.experimental.pallas import tpu as pltpu
from jax.experimental.pallas import tpu_sc as plsc

N = 100000
D_ATOM = 32
D_RES = 16
D_AR = D_ATOM + D_RES

NC, NS = 2, 16
NW = NC * NS
SUB = 80
NSUB = 5
C = SUB * NSUB
NG = C // 16
NCHUNKS = N // C
BASE_CHUNKS = NCHUNKS // NW
BIG = NCHUNKS - NW * BASE_CHUNKS
MAXC = BASE_CHUNKS + 1


def _body(atom_idx_hbm, res_idx_hbm, wcat_hbm, outT_hbm,
          idx_a, idx_r, bufs0, bufs1, sem0, sem1, semw0, semw1):
    wid = lax.axis_index("s") * NC + lax.axis_index("c")
    start = wid * BASE_CHUNKS + jnp.minimum(wid, BIG)
    count = jnp.where(wid < BIG, BASE_CHUNKS + 1, BASE_CHUNKS)
    end = start + count
    base0 = start * C
    lanes = lax.iota(jnp.int32, 16)

    pltpu.sync_copy(atom_idx_hbm.at[pl.ds(base0, BASE_CHUNKS * C)],
                    idx_a.at[pl.ds(0, BASE_CHUNKS * C)])
    pltpu.sync_copy(res_idx_hbm.at[pl.ds(base0, BASE_CHUNKS * C)],
                    idx_r.at[pl.ds(0, BASE_CHUNKS * C)])

    @pl.when(count == MAXC)
    def _():
        pltpu.sync_copy(
            atom_idx_hbm.at[pl.ds(base0 + BASE_CHUNKS * C, C)],
            idx_a.at[pl.ds(BASE_CHUNKS * C, C)])
        pltpu.sync_copy(
            res_idx_hbm.at[pl.ds(base0 + BASE_CHUNKS * C, C)],
            idx_r.at[pl.ds(BASE_CHUNKS * C, C)])

    def fire(k, bufs, sem):
        rows_a, rows_r, _ = bufs
        off = (k - start) * C
        for j in range(NSUB):
            pltpu.async_copy(wcat_hbm.at[idx_a.at[pl.ds(off + SUB * j, SUB)]],
                             rows_a.at[pl.ds(SUB * j, SUB)], sem)
            pltpu.async_copy(wcat_hbm.at[idx_r.at[pl.ds(off + SUB * j, SUB)]],
                             rows_r.at[pl.ds(SUB * j, SUB)], sem)

    def drain(bufs, sem):
        rows_a, rows_r, _ = bufs
        for j in range(NSUB):
            pltpu.make_async_copy(wcat_hbm.at[pl.ds(0, SUB)],
                                  rows_a.at[pl.ds(SUB * j, SUB)], sem).wait()
            pltpu.make_async_copy(wcat_hbm.at[pl.ds(0, SUB)],
                                  rows_r.at[pl.ds(SUB * j, SUB)], sem).wait()

    def proc(k, bufs, semw):
        rows_a, rows_r, outT_v = bufs
        base = k * C

        @pl.when(k >= start + 2)
        def _():
            pltpu.make_async_copy(outT_v, outT_hbm.at[:, pl.ds(0, C)],
                                  semw).wait()

        @plsc.parallel_loop(0, NG)
        def col_group(g):
            rows16 = g * 16 + lanes
            for c in range(D_ATOM):
                cols16 = jnp.full((16,), c, jnp.int32)
                outT_v[c, pl.ds(g * 16, 16)] = plsc.load_gather(
                    rows_a, [rows16, cols16])
            for c in range(D_RES):
                cols16 = jnp.full((16,), D_ATOM + c, jnp.int32)
                outT_v[D_ATOM + c, pl.ds(g * 16, 16)] = plsc.load_gather(
                    rows_r, [rows16, cols16])

        pltpu.async_copy(outT_v, outT_hbm.at[:, pl.ds(base, C)], semw)

    fire(start, bufs0, sem0)

    def pair_body(p, carry):
        k0 = start + 2 * p
        k1 = k0 + 1

        @pl.when(k1 < end)
        def _():
            fire(k1, bufs1, sem1)

        drain(bufs0, sem0)
        proc(k0, bufs0, semw0)

        @pl.when(k1 < end)
        def _():
            @pl.when(k1 + 1 < end)
            def _():
                fire(k1 + 1, bufs0, sem0)

            drain(bufs1, sem1)
            proc(k1, bufs1, semw1)

        return carry

    lax.fori_loop(0, (MAXC + 1) // 2, pair_body, 0)

    pltpu.make_async_copy(bufs0[2], outT_hbm.at[:, pl.ds(0, C)],
                          semw0).wait()
    pltpu.make_async_copy(bufs1[2], outT_hbm.at[:, pl.ds(0, C)],
                          semw1).wait()


def _buf_types():
    return (
        pltpu.VMEM((C, D_AR), jnp.float32),
        pltpu.VMEM((C, D_AR), jnp.float32),
        pltpu.VMEM((D_AR, C), jnp.float32),
    )


@jax.jit
def _sc_embed(atom_types, residue_types, W_cat):
    mesh = plsc.VectorSubcoreMesh(core_axis_name="c", subcore_axis_name="s",
                                  num_cores=NC, num_subcores=NS)
    f = functools.partial(
        pl.kernel,
        out_type=jax.ShapeDtypeStruct((D_AR, N), jnp.float32),
        mesh=mesh,
        scratch_types=[
            pltpu.VMEM((MAXC * C,), jnp.int32),
            pltpu.VMEM((MAXC * C,), jnp.int32),
            _buf_types(),
            _buf_types(),
            pltpu.SemaphoreType.DMA,
            pltpu.SemaphoreType.DMA,
            pltpu.SemaphoreType.DMA,
            pltpu.SemaphoreType.DMA,
        ],
        compiler_params=pltpu.CompilerParams(use_tc_tiling_on_sc=False,
                                             needs_layout_passes=False),
    )(_body)
    return f(atom_types, residue_types, W_cat)


def kernel(atom_types, residue_types, extra_feats, W_atom, W_res):
    W_cat = jnp.concatenate([W_atom, W_res], axis=1)
    outT_ar = _sc_embed(atom_types, residue_types, W_cat)
    return jnp.concatenate([outT_ar.T, extra_feats], axis=1)
